# baseline (speedup 1.0000x reference)
import functools

import jax
import jax.numpy as jnp
from jax import lax
from jax.experimental import pallas as pl
from jax.experimental.pallas import tpu as pltpu
from jax.experimental.pallas import tpu_sc as plsc

_ROWS = 1000000
_COLS = 64
_B = 16384
_N = _B * _COLS
_NC = 2
_NS = 16
_NW = _NC * _NS
_PER_W = _N // _NW
_CHUNK = 128
_NCH = _PER_W // _CHUNK
_FIRE = 8
_L = 16


def _body(data_hbm, idx_hbm, out_hbm, idx_v, val_v, sem):
    wid = lax.axis_index("s") * _NC + lax.axis_index("c")

    pltpu.sync_copy(idx_hbm.at[wid], idx_v)

    lane = lax.iota(jnp.int32, _L)

    @pl.loop(0, _NCH)
    def _flat(ch):
        for s in range(_CHUNK // _L):
            col = (s % 4) * _L
            sl = (ch, pl.ds(s * _L, _L))
            idx_v[sl] = idx_v[sl] * _COLS + (lane + col)

    @pl.loop(0, _NCH // _FIRE)
    def _gather(g):
        descs = []
        for b in range(_FIRE):
            ch = g * _FIRE + b
            descs.append(
                pltpu.async_copy(
                    data_hbm.at[idx_v.at[ch]],
                    val_v.at[ch],
                    sem,
                )
            )
        for d in descs:
            d.wait()

    @pl.loop(0, _NCH)
    def _double(ch):
        for s in range(_CHUNK // _L):
            sl = (ch, pl.ds(s * _L, _L))
            val_v[sl] = val_v[sl] * 2.0

    pltpu.sync_copy(val_v, out_hbm.at[wid])


@jax.jit
def _run(data_flat, idx3):
    mesh = plsc.VectorSubcoreMesh(core_axis_name="c", subcore_axis_name="s")
    k = functools.partial(
        pl.kernel,
        out_type=jax.ShapeDtypeStruct((_NW, _NCH, _CHUNK), jnp.float32),
        mesh=mesh,
        scratch_types=[
            pltpu.VMEM((_NCH, _CHUNK), jnp.int32),
            pltpu.VMEM((_NCH, _CHUNK), jnp.float32),
            pltpu.SemaphoreType.DMA,
        ],
    )(_body)
    return k(data_flat, idx3)


def kernel(data, indices):
    data_flat = data.reshape(_ROWS * _COLS)
    idx3 = indices.astype(jnp.int32).reshape(_NW, _NCH, _CHUNK)
    out = _run(data_flat, idx3)
    return out.reshape(_B, _COLS)
